# baseline (speedup 1.0000x reference)
import functools

import jax
import jax.numpy as jnp
from jax import lax
from jax.experimental import pallas as pl
from jax.experimental.pallas import tpu as pltpu

N = 1000000
IN_CH = 10
HALF = 32
NSEG = 30000
EPS = 1e-3

BR_STATS = 20000
BR_MAIN = 10000


def _stats_body(x_ref, s_ref, g_ref):
    step = pl.program_id(0)
    blk = x_ref[...]

    @pl.when(step == 0)
    def _():
        s_ref[...] = jnp.zeros_like(s_ref)
        g_ref[...] = jnp.zeros_like(g_ref)

    s_ref[...] += jnp.sum(blk, axis=0, keepdims=True)
    g_ref[...] += lax.dot_general(blk, blk, (((0,), (0,)), ((), ())),
                                  preferred_element_type=jnp.float32)


def _main_body(x_ref, w_ref, b_ref, o_ref):
    o_ref[...] = jnp.maximum(
        lax.dot_general(x_ref[...], w_ref[...], (((1,), (1,)), ((), ())),
                        preferred_element_type=jnp.float32) + b_ref[...],
        0.0)


def _bn_folded_weights(inputs, W, gamma, beta):
    s, g = pl.pallas_call(
        _stats_body,
        grid=(N // BR_STATS,),
        in_specs=[pl.BlockSpec((BR_STATS, IN_CH), lambda i: (i, 0))],
        out_specs=[pl.BlockSpec((1, IN_CH), lambda i: (0, 0)),
                   pl.BlockSpec((IN_CH, IN_CH), lambda i: (0, 0))],
        out_shape=[jax.ShapeDtypeStruct((1, IN_CH), jnp.float32),
                   jax.ShapeDtypeStruct((IN_CH, IN_CH), jnp.float32)],
    )(inputs)
    mu = s[0] / N
    second = g / N
    mean_x = W @ mu
    e2 = jnp.sum((W @ second) * W, axis=1)
    var_x = e2 - mean_x * mean_x
    scale = gamma / jnp.sqrt(var_x + EPS)
    Wp = W * scale[:, None]
    bp = beta - mean_x * scale
    return Wp, bp


def _linear_relu(inputs, Wp, bp):
    return pl.pallas_call(
        _main_body,
        grid=(N // BR_MAIN,),
        in_specs=[pl.BlockSpec((BR_MAIN, IN_CH), lambda i: (i, 0)),
                  pl.BlockSpec((HALF, IN_CH), lambda i: (0, 0)),
                  pl.BlockSpec((1, HALF), lambda i: (0, 0))],
        out_specs=pl.BlockSpec((BR_MAIN, HALF), lambda i: (i, 0)),
        out_shape=jax.ShapeDtypeStruct((N, HALF), jnp.float32),
    )(inputs, Wp, bp.reshape(1, HALF))


def kernel(inputs, unq_inv, W, gamma, beta):
    Wp, bp = _bn_folded_weights(inputs, W, gamma, beta)
    x = _linear_relu(inputs, Wp, bp)
    x_max = jax.ops.segment_max(x, unq_inv, num_segments=NSEG,
                                indices_are_sorted=True)
    return jnp.concatenate([x, x_max[unq_inv, :]], axis=1)


# speedup vs baseline: 1.3998x; 1.3998x over previous
import functools

import jax
import jax.numpy as jnp
from jax import lax
from jax.experimental import pallas as pl
from jax.experimental.pallas import tpu as pltpu
from jax.experimental.pallas import tpu_sc as plsc

N = 1000000
IN_CH = 10
HALF = 32
NSEG = 30000
EPS = 1e-3

BR_STATS = 20000
BR_MAIN = 10000

NW = 32
SEG_PER_W = 960
NSEG_PAD = SEG_PER_W * NW
RS_DMA = 976
RS_LEN = (NW - 1) * SEG_PER_W + RS_DMA
CH = 512
SEGB = 64

RPW = 31232
GCH = 976
TAIL = N - RPW * NW


def _stats_body(x_ref, s_ref, g_ref):
    step = pl.program_id(0)
    blk = x_ref[...]

    @pl.when(step == 0)
    def _():
        s_ref[...] = jnp.zeros_like(s_ref)
        g_ref[...] = jnp.zeros_like(g_ref)

    s_ref[...] += jnp.sum(blk, axis=0, keepdims=True)
    g_ref[...] += lax.dot_general(blk, blk, (((0,), (0,)), ((), ())),
                                  preferred_element_type=jnp.float32)


def _main_body(x_ref, w_ref, b_ref, o_ref):
    o_ref[...] = jnp.maximum(
        lax.dot_general(x_ref[...], w_ref[...], (((1,), (1,)), ((), ())),
                        preferred_element_type=jnp.float32) + b_ref[...],
        0.0)


def _bn_folded_weights(inputs, W, gamma, beta):
    s, g = pl.pallas_call(
        _stats_body,
        grid=(N // BR_STATS,),
        in_specs=[pl.BlockSpec((BR_STATS, IN_CH), lambda i: (i, 0))],
        out_specs=[pl.BlockSpec((1, IN_CH), lambda i: (0, 0)),
                   pl.BlockSpec((IN_CH, IN_CH), lambda i: (0, 0))],
        out_shape=[jax.ShapeDtypeStruct((1, IN_CH), jnp.float32),
                   jax.ShapeDtypeStruct((IN_CH, IN_CH), jnp.float32)],
    )(inputs)
    mu = s[0] / N
    second = g / N
    mean_x = W @ mu
    e2 = jnp.sum((W @ second) * W, axis=1)
    var_x = e2 - mean_x * mean_x
    scale = gamma / jnp.sqrt(var_x + EPS)
    Wp = W * scale[:, None]
    bp = beta - mean_x * scale
    return Wp, bp


def _linear_relu(inputs, Wp, bp):
    return pl.pallas_call(
        _main_body,
        grid=(N // BR_MAIN,),
        in_specs=[pl.BlockSpec((BR_MAIN, IN_CH), lambda i: (i, 0)),
                  pl.BlockSpec((HALF, IN_CH), lambda i: (0, 0)),
                  pl.BlockSpec((1, HALF), lambda i: (0, 0))],
        out_specs=pl.BlockSpec((BR_MAIN, HALF), lambda i: (i, 0)),
        out_shape=jax.ShapeDtypeStruct((N, HALF), jnp.float32),
    )(inputs, Wp, bp.reshape(1, HALF))


_SC_MESH = plsc.VectorSubcoreMesh(core_axis_name="c", subcore_axis_name="s")


@functools.partial(
    pl.kernel,
    out_type=jax.ShapeDtypeStruct((NSEG_PAD, HALF), jnp.float32),
    mesh=_SC_MESH,
    compiler_params=pltpu.CompilerParams(use_tc_tiling_on_sc=False),
    scratch_types=[
        pltpu.VMEM((RS_DMA,), jnp.int32),
        pltpu.VMEM((CH, HALF), jnp.float32),
        pltpu.VMEM((SEGB, HALF), jnp.float32),
        pltpu.SemaphoreType.DMA,
        pltpu.SemaphoreType.DMA,
    ],
)
def _segmax_kernel(x_hbm, rs_hbm, xmax_hbm, rs_v, xbuf, xmax_v, sem, semf):
    wid = lax.axis_index("s") * 2 + lax.axis_index("c")
    seg_lo = pl.multiple_of(wid * SEG_PER_W, 8)
    pltpu.sync_copy(rs_hbm.at[pl.ds(seg_lo, RS_DMA)], rs_v)
    r0 = rs_v[pl.ds(0, 16)][0]
    zero16 = jnp.zeros((16,), jnp.float32)

    def stage(r):
        wl = pl.multiple_of(jnp.minimum((r // 8) * 8, N - CH), 8)
        pltpu.async_copy(x_hbm.at[pl.ds(wl, CH)], xbuf, sem).wait()
        return wl

    win0 = stage(r0)

    def seg_body(si, win_lo):
        bounds = rs_v[pl.ds(si, 16)]
        a, b = bounds[0], bounds[1]

        def row_body(r, c):
            a0, a1, wl = c
            need = r >= wl + CH
            new_wl = pl.multiple_of(jnp.where(need, stage_val(r), wl), 8)

            @pl.when(need)
            def _():
                pltpu.async_copy(x_hbm.at[pl.ds(new_wl, CH)], xbuf, sem).wait()

            off = r - new_wl
            return (jnp.maximum(a0, xbuf[off, pl.ds(0, 16)]),
                    jnp.maximum(a1, xbuf[off, pl.ds(16, 16)]),
                    new_wl)

        def stage_val(r):
            return pl.multiple_of(jnp.minimum((r // 8) * 8, N - CH), 8)

        a0, a1, win_lo = lax.fori_loop(a, b, row_body,
                                       (zero16, zero16, win_lo))
        slot = lax.rem(si, SEGB)
        xmax_v[slot, pl.ds(0, 16)] = a0
        xmax_v[slot, pl.ds(16, 16)] = a1

        @pl.when(slot == SEGB - 1)
        def _():
            dst = pl.multiple_of(seg_lo + si - (SEGB - 1), 8)
            pltpu.async_copy(xmax_v, xmax_hbm.at[pl.ds(dst, SEGB)],
                             semf).wait()

        return win_lo

    lax.fori_loop(0, SEG_PER_W, seg_body, win0)


@functools.partial(
    pl.kernel,
    out_type=jax.ShapeDtypeStruct((N, HALF), jnp.float32),
    mesh=_SC_MESH,
    compiler_params=pltpu.CompilerParams(use_tc_tiling_on_sc=False),
    scratch_types=[
        pltpu.VMEM((GCH,), jnp.int32),
        pltpu.VMEM((GCH, HALF), jnp.float32),
        pltpu.SemaphoreType.DMA,
        pltpu.SemaphoreType.DMA,
    ],
)
def _gather_kernel(idx_hbm, xmax_hbm, out_hbm, idx_v, gbuf, sem, sem2):
    wid = lax.axis_index("s") * 2 + lax.axis_index("c")
    base = pl.multiple_of(wid * RPW, 8)

    def do_chunk(row, nrows):
        pltpu.async_copy(idx_hbm.at[pl.ds(row, nrows)],
                         idx_v.at[pl.ds(0, nrows)], sem2).wait()
        pltpu.async_copy(xmax_hbm.at[idx_v.at[pl.ds(0, nrows)]],
                         gbuf.at[pl.ds(0, nrows)], sem).wait()
        pltpu.async_copy(gbuf.at[pl.ds(0, nrows)],
                         out_hbm.at[pl.ds(row, nrows)], sem2).wait()

    def chunk_body(k, _):
        do_chunk(pl.multiple_of(base + k * GCH, 8), GCH)
        return 0

    lax.fori_loop(0, RPW // GCH, chunk_body, 0)

    @pl.when(wid == 0)
    def _():
        do_chunk(RPW * NW, TAIL)


def kernel(inputs, unq_inv, W, gamma, beta):
    Wp, bp = _bn_folded_weights(inputs, W, gamma, beta)
    x = _linear_relu(inputs, Wp, bp)
    unq32 = unq_inv.astype(jnp.int32)
    rs = jnp.searchsorted(unq32, jnp.arange(RS_LEN, dtype=jnp.int32),
                          side="left").astype(jnp.int32)
    x_max = _segmax_kernel(x, rs)
    gathered = _gather_kernel(unq32, x_max)
    return jnp.concatenate([x, gathered], axis=1)
